# speedup vs baseline: 1.0488x; 1.0488x over previous
import jax
import jax.numpy as jnp
from jax.experimental import pallas as pl

N_GRAPHS = 512


def _mlp(layers, x):
    for i, p in enumerate(layers):
        x = x @ p['W'] + p['b']
        if i < len(layers) - 1:
            x = jax.nn.relu(x)
    return x


def _seg_mean(data, ids, n):
    s = jax.ops.segment_sum(data, ids, num_segments=n)
    c = jax.ops.segment_sum(jnp.ones((data.shape[0], 1), data.dtype), ids, num_segments=n)
    return s / jnp.maximum(c, 1.0)


def _block_apply(p, v, edge_index, e, u, batch, n_graphs):
    v = _mlp(p['node_pre'], v)
    e = _mlp(p['edge_pre'], e)
    u = _mlp(p['state_pre'], u)
    src = edge_index[0]
    dst = edge_index[1]
    eb = batch[src]
    e_new = _mlp(p['phi_e'], jnp.concatenate([v[src], v[dst], e, u[eb]], axis=1))
    agg = _seg_mean(e_new, dst, v.shape[0])
    v_new = _mlp(p['phi_v'], jnp.concatenate([v, agg, u[batch]], axis=1))
    ue = _seg_mean(e_new, eb, n_graphs)
    uv = _seg_mean(v_new, batch, n_graphs)
    u_new = _mlp(p['phi_u'], jnp.concatenate([ue, uv, u], axis=1))
    return v_new, e_new, u_new


def _set2set(p, x, ids, n, steps=3):
    c = p['W_hh'].shape[0]
    h = jnp.zeros((n, c), dtype=x.dtype)
    cell = jnp.zeros((n, c), dtype=x.dtype)
    q_star = jnp.zeros((n, 2 * c), dtype=x.dtype)
    for _ in range(steps):
        gates = q_star @ p['W_ih'] + h @ p['W_hh'] + p['b']
        i, f, g, o = jnp.split(gates, 4, axis=1)
        cell = jax.nn.sigmoid(f) * cell + jax.nn.sigmoid(i) * jnp.tanh(g)
        h = jax.nn.sigmoid(o) * jnp.tanh(cell)
        q = h
        energy = jnp.sum(x * q[ids], axis=1)
        emax = jax.ops.segment_max(energy, ids, num_segments=n)
        emax = jnp.where(jnp.isfinite(emax), emax, 0.0)
        ex = jnp.exp(energy - emax[ids])
        denom = jax.ops.segment_sum(ex, ids, num_segments=n)
        a = ex / (denom[ids] + 1e-16)
        r = jax.ops.segment_sum(a[:, None] * x, ids, num_segments=n)
        q_star = jnp.concatenate([q, r], axis=1)
    return q_star


def _rbf(d, n_centers=100):
    mu = jnp.linspace(0.0, 5.0, n_centers)
    return jnp.exp(-((d[:, None] - mu[None, :]) ** 2) / (0.5 ** 2))


def _head_kernel(z_ref, w1_ref, b1_ref, g1_ref, be1_ref, w2_ref, b2_ref,
                 g2_ref, be2_ref, w3_ref, b3_ref, out_ref):
    z = z_ref[...]

    def bn(t, gamma, beta):
        m = jnp.mean(t, axis=0, keepdims=True)
        v = jnp.mean((t - m) ** 2, axis=0, keepdims=True)
        return (t - m) / jnp.sqrt(v + 1e-5) * gamma + beta

    h1 = z @ w1_ref[...] + b1_ref[...]
    h1 = jax.nn.relu(bn(h1, g1_ref[...], be1_ref[...]))
    h2 = h1 @ w2_ref[...] + b2_ref[...]
    h2 = jax.nn.relu(bn(h2, g2_ref[...], be2_ref[...]))
    out_ref[...] = h2 @ w3_ref[...] + b3_ref[...]


def _head(hd, z):
    args = (z, hd['W1'], hd['b1'].reshape(1, -1), hd['g1'].reshape(1, -1),
            hd['be1'].reshape(1, -1), hd['W2'], hd['b2'].reshape(1, -1),
            hd['g2'].reshape(1, -1), hd['be2'].reshape(1, -1), hd['W3'],
            hd['b3'].reshape(1, -1))
    return pl.pallas_call(
        _head_kernel,
        out_shape=jax.ShapeDtypeStruct((z.shape[0], 1), jnp.float32),
    )(*args)


def kernel(x, edge_index, edge_attr, state, batch, params):
    v = params['emb'][x]
    e = _rbf(edge_attr)
    v, e, u = _block_apply(params['block1'], v, edge_index, e, state, batch, N_GRAPHS)
    v, e, u = _block_apply(params['block2'], v, edge_index, e, u, batch, N_GRAPHS)
    xg = _set2set(params['s2s_nodes'], v, batch, N_GRAPHS)
    eb = batch[edge_index[0]]
    eg = _set2set(params['s2s_edges'], e, eb, N_GRAPHS)
    z = jnp.concatenate([xg, eg, u], axis=1)
    return _head(params['head'], z)


# speedup vs baseline: 3.1509x; 3.0043x over previous
import functools

import jax
import jax.numpy as jnp
from jax import lax
from jax.experimental import pallas as pl
from jax.experimental.pallas import tpu as pltpu

F32 = jnp.float32

B = 2000


def _dot(a, b):
    return jnp.dot(a, b, preferred_element_type=F32)


def _dott(a, b):
    return lax.dot_general(a, b, (((1,), (1,)), ((), ())),
                           preferred_element_type=F32)


def _dotl(a, b):
    return lax.dot_general(a, b, (((0,), (0,)), ((), ())),
                           preferred_element_type=F32)


def _row(w):
    return pl.BlockSpec((B, w), lambda i: (i, 0))


def _sca():
    return pl.BlockSpec((1, 1, B), lambda i: (i, 0, 0))


def _full(*s):
    return pl.BlockSpec(s, lambda i: (0,) * len(s))


def _k1_body(x_ref, emb_ref, w1_ref, b1_ref, w2_ref, b2_ref, ws_ref, wd_ref,
             v1_ref, as_ref, ad_ref):
    xrow = x_ref[0]
    oht = (xrow == lax.broadcasted_iota(jnp.int32, (100, B), 0)).astype(F32)
    v0 = _dotl(oht, emb_ref[...])
    h = jax.nn.relu(_dot(v0, w1_ref[...]) + b1_ref[...])
    v1 = _dot(h, w2_ref[...]) + b2_ref[...]
    v1_ref[...] = v1
    as_ref[...] = _dot(v1, ws_ref[...])
    ad_ref[...] = _dot(v1, wd_ref[...])


def _k1(x3, emb, npre, ws, wd, n):
    return pl.pallas_call(
        _k1_body,
        grid=(n // B,),
        in_specs=[_sca(), _full(100, 64), _full(64, 64), _full(1, 64),
                  _full(64, 32), _full(1, 32), _full(32, 64), _full(32, 64)],
        out_specs=[_row(32), _row(64), _row(64)],
        out_shape=[jax.ShapeDtypeStruct((n, 32), F32),
                   jax.ShapeDtypeStruct((n, 64), F32),
                   jax.ShapeDtypeStruct((n, 64), F32)],
    )(x3, emb, npre[0]['W'], npre[0]['b'].reshape(1, -1),
      npre[1]['W'], npre[1]['b'].reshape(1, -1), ws, wd)


def _kstate_body(u_ref, w1_ref, b1_ref, w2_ref, b2_ref, wu_ref, bphi_ref,
                 up_ref, c_ref):
    h = jax.nn.relu(_dot(u_ref[...], w1_ref[...]) + b1_ref[...])
    up = _dot(h, w2_ref[...]) + b2_ref[...]
    up_ref[...] = up
    c_ref[...] = _dot(up, wu_ref[...]) + bphi_ref[...]


def _kstate(u, spre, wu, bphi):
    g = u.shape[0]
    return pl.pallas_call(
        _kstate_body,
        out_shape=[jax.ShapeDtypeStruct((g, 32), F32),
                   jax.ShapeDtypeStruct((g, 64), F32)],
    )(u, spre[0]['W'], spre[0]['b'].reshape(1, -1),
      spre[1]['W'], spre[1]['b'].reshape(1, -1), wu, bphi.reshape(1, -1))


def _k34_body(attr_ref, gsum_ref, w1_ref, b1_ref, w2_ref, b2_ref, we1_ref,
              pw2_ref, pb2_ref, q1_ref, qb1_ref, q2_ref, qb2_ref, we2_ref,
              e1_ref, t2_ref):
    drow = attr_ref[0]
    mu = lax.broadcasted_iota(jnp.int32, (100, B), 0).astype(F32) * F32(5.0 / 99.0)
    pt = jnp.exp(-((mu - drow) ** 2) * F32(4.0))
    h = jax.nn.relu(_dotl(pt, w1_ref[...]) + b1_ref[...])
    epre = _dot(h, w2_ref[...]) + b2_ref[...]
    hh = jax.nn.relu(_dot(epre, we1_ref[...]) + gsum_ref[...])
    e1 = _dot(hh, pw2_ref[...]) + pb2_ref[...]
    e1_ref[...] = e1
    h2 = jax.nn.relu(_dot(e1, q1_ref[...]) + qb1_ref[...])
    ep2 = _dot(h2, q2_ref[...]) + qb2_ref[...]
    t2_ref[...] = _dot(ep2, we2_ref[...])


def _k34(attr3, gsum1, epre1, we1, phi1_w2, phi1_b2, epre2, we2, e):
    return pl.pallas_call(
        _k34_body,
        grid=(e // B,),
        in_specs=[_sca(), _row(64), _full(100, 64), _full(1, 64),
                  _full(64, 32), _full(1, 32), _full(32, 64), _full(64, 32),
                  _full(1, 32), _full(32, 64), _full(1, 64), _full(64, 32),
                  _full(1, 32), _full(32, 64)],
        out_specs=[_row(32), _row(64)],
        out_shape=[jax.ShapeDtypeStruct((e, 32), F32),
                   jax.ShapeDtypeStruct((e, 64), F32)],
    )(attr3, gsum1, epre1[0]['W'], epre1[0]['b'].reshape(1, -1),
      epre1[1]['W'], epre1[1]['b'].reshape(1, -1), we1, phi1_w2,
      phi1_b2.reshape(1, -1), epre2[0]['W'], epre2[0]['b'].reshape(1, -1),
      epre2[1]['W'], epre2[1]['b'].reshape(1, -1), we2)


def _k7_body(t2_ref, gsum_ref, w2_ref, b2_ref, e2_ref):
    hh = jax.nn.relu(t2_ref[...] + gsum_ref[...])
    e2_ref[...] = _dot(hh, w2_ref[...]) + b2_ref[...]


def _k7(t2, gsum2, w2, b2, e):
    return pl.pallas_call(
        _k7_body,
        grid=(e // B,),
        in_specs=[_row(64), _row(64), _full(64, 32), _full(1, 32)],
        out_specs=_row(32),
        out_shape=jax.ShapeDtypeStruct((e, 32), F32),
    )(t2, gsum2, w2, b2.reshape(1, -1))


def _k5_body(v_ref, acc_ref, cnt_ref, b_ref, up_ref, p1_ref, p2_ref, p3_ref,
             pb1_ref, pw2_ref, pb2_ref, n1_ref, nb1_ref, n2_ref, nb2_ref,
             ws_ref, wd_ref, v2p_ref, as_ref, ad_ref, uv_ref, *, ng):
    i = pl.program_id(0)
    agg = acc_ref[...] / jnp.maximum(cnt_ref[...], F32(1.0))
    brow = b_ref[0]
    oht = (brow == lax.broadcasted_iota(jnp.int32, (ng, B), 0)).astype(F32)
    ub = _dotl(oht, up_ref[...])
    h = jax.nn.relu(_dot(v_ref[...], p1_ref[...]) + _dot(agg, p2_ref[...])
                    + _dot(ub, p3_ref[...]) + pb1_ref[...])
    vnew = _dot(h, pw2_ref[...]) + pb2_ref[...]
    hn = jax.nn.relu(_dot(vnew, n1_ref[...]) + nb1_ref[...])
    v2p = _dot(hn, n2_ref[...]) + nb2_ref[...]
    v2p_ref[...] = v2p
    as_ref[...] = _dot(v2p, ws_ref[...])
    ad_ref[...] = _dot(v2p, wd_ref[...])
    ones = jnp.ones((B, 1), F32)
    upd = _dot(oht, jnp.concatenate([ones, vnew], axis=1))

    @pl.when(i == 0)
    def _():
        uv_ref[...] = jnp.zeros_like(uv_ref)

    uv_ref[...] += upd


def _k5(v1, nodeacc, u1p, batch3, phiv, npre2, ws2, wd2, n, ng):
    w1 = phiv[0]['W']
    return pl.pallas_call(
        functools.partial(_k5_body, ng=ng),
        grid=(n // B,),
        in_specs=[_row(32), _row(32), _row(1), _sca(), _full(ng, 32),
                  _full(32, 64), _full(32, 64), _full(32, 64), _full(1, 64),
                  _full(64, 32), _full(1, 32), _full(32, 64), _full(1, 64),
                  _full(64, 32), _full(1, 32), _full(32, 64), _full(32, 64)],
        out_specs=[_row(32), _row(64), _row(64),
                   pl.BlockSpec((ng, 33), lambda i: (0, 0))],
        out_shape=[jax.ShapeDtypeStruct((n, 32), F32),
                   jax.ShapeDtypeStruct((n, 64), F32),
                   jax.ShapeDtypeStruct((n, 64), F32),
                   jax.ShapeDtypeStruct((ng, 33), F32)],
    )(v1, nodeacc[:, 1:], nodeacc[:, 0:1], batch3, u1p,
      w1[0:32], w1[32:64], w1[64:96], phiv[0]['b'].reshape(1, -1),
      phiv[1]['W'], phiv[1]['b'].reshape(1, -1), npre2[0]['W'],
      npre2[0]['b'].reshape(1, -1), npre2[1]['W'],
      npre2[1]['b'].reshape(1, -1), ws2, wd2)


def _k8_body(v_ref, acc_ref, cnt_ref, b_ref, up_ref, p1_ref, p2_ref, p3_ref,
             pb1_ref, pw2_ref, pb2_ref, v2_ref, uv_ref, *, ng):
    i = pl.program_id(0)
    agg = acc_ref[...] / jnp.maximum(cnt_ref[...], F32(1.0))
    brow = b_ref[0]
    oht = (brow == lax.broadcasted_iota(jnp.int32, (ng, B), 0)).astype(F32)
    ub = _dotl(oht, up_ref[...])
    h = jax.nn.relu(_dot(v_ref[...], p1_ref[...]) + _dot(agg, p2_ref[...])
                    + _dot(ub, p3_ref[...]) + pb1_ref[...])
    vnew = _dot(h, pw2_ref[...]) + pb2_ref[...]
    v2_ref[...] = vnew
    ones = jnp.ones((B, 1), F32)
    upd = _dot(oht, jnp.concatenate([ones, vnew], axis=1))

    @pl.when(i == 0)
    def _():
        uv_ref[...] = jnp.zeros_like(uv_ref)

    uv_ref[...] += upd


def _k8(v2p, nodeacc, u2p, batch3, phiv, n, ng):
    w1 = phiv[0]['W']
    return pl.pallas_call(
        functools.partial(_k8_body, ng=ng),
        grid=(n // B,),
        in_specs=[_row(32), _row(32), _row(1), _sca(), _full(ng, 32),
                  _full(32, 64), _full(32, 64), _full(32, 64), _full(1, 64),
                  _full(64, 32), _full(1, 32)],
        out_specs=[_row(32), pl.BlockSpec((ng, 33), lambda i: (0, 0))],
        out_shape=[jax.ShapeDtypeStruct((n, 32), F32),
                   jax.ShapeDtypeStruct((ng, 33), F32)],
    )(v2p, nodeacc[:, 1:], nodeacc[:, 0:1], batch3, u2p,
      w1[0:32], w1[32:64], w1[64:96], phiv[0]['b'].reshape(1, -1),
      phiv[1]['W'], phiv[1]['b'].reshape(1, -1))


def _kphiu_body(gacc_ref, uvacc_ref, u_ref, p1_ref, p2_ref, p3_ref, pb1_ref,
                pw2_ref, pb2_ref, unew_ref):
    ue = gacc_ref[:, 1:33] / jnp.maximum(gacc_ref[:, 0:1], F32(1.0))
    uv = uvacc_ref[:, 1:33] / jnp.maximum(uvacc_ref[:, 0:1], F32(1.0))
    h = jax.nn.relu(_dot(ue, p1_ref[...]) + _dot(uv, p2_ref[...])
                    + _dot(u_ref[...], p3_ref[...]) + pb1_ref[...])
    unew_ref[...] = _dot(h, pw2_ref[...]) + pb2_ref[...]


def _kphiu(gacc, uvacc, u, phiu, ng):
    w1 = phiu[0]['W']
    return pl.pallas_call(
        _kphiu_body,
        out_shape=jax.ShapeDtypeStruct((ng, 32), F32),
    )(gacc, uvacc, u, w1[0:32], w1[32:64], w1[64:96],
      phiu[0]['b'].reshape(1, -1), phiu[1]['W'], phiu[1]['b'].reshape(1, -1))


def _s2sl_body(h_ref, c_ref, q_ref, gacc_ref, wih_ref, whh_ref, bias_ref,
               ho_ref, co_ref, qo_ref):
    r = gacc_ref[:, 1:33] / (gacc_ref[:, 0:1] + F32(1e-16))
    q_star = jnp.concatenate([q_ref[...], r], axis=1)
    gates = (_dot(q_star, wih_ref[...]) + _dot(h_ref[...], whh_ref[...])
             + bias_ref[...])
    ii, ff, gg, oo = jnp.split(gates, 4, axis=1)
    cell = jax.nn.sigmoid(ff) * c_ref[...] + jax.nn.sigmoid(ii) * jnp.tanh(gg)
    h = jax.nn.sigmoid(oo) * jnp.tanh(cell)
    ho_ref[...] = h
    co_ref[...] = cell
    qo_ref[...] = h


def _s2sl(h, c, q, gacc, p, ng):
    return pl.pallas_call(
        _s2sl_body,
        out_shape=[jax.ShapeDtypeStruct((ng, 32), F32)] * 3,
    )(h, c, q, gacc, p['W_ih'], p['W_hh'], p['b'].reshape(1, -1))


def _s2sa_body(d_ref, ids_ref, q_ref, en_ref, emax_ref, *, ng):
    i = pl.program_id(0)
    eallt = _dott(q_ref[...], d_ref[...])
    idrow = ids_ref[0]
    oht = idrow == lax.broadcasted_iota(jnp.int32, (ng, B), 0)
    en_ref[0] = jnp.sum(jnp.where(oht, eallt, F32(0.0)), axis=0, keepdims=True)
    loc = jnp.max(jnp.where(oht, eallt, F32(-jnp.inf)), axis=1, keepdims=True)

    @pl.when(i == 0)
    def _():
        emax_ref[...] = jnp.full_like(emax_ref, -jnp.inf)

    emax_ref[...] = jnp.maximum(emax_ref[...], loc)


def _s2sa(data, ids3, q, rows, ng):
    return pl.pallas_call(
        functools.partial(_s2sa_body, ng=ng),
        grid=(rows // B,),
        in_specs=[_row(32), _sca(), _full(ng, 32)],
        out_specs=[_sca(), pl.BlockSpec((ng, 1), lambda i: (0, 0))],
        out_shape=[jax.ShapeDtypeStruct((rows // B, 1, B), F32),
                   jax.ShapeDtypeStruct((ng, 1), F32)],
    )(data, ids3, q)


def _s2sb_body(d_ref, ids_ref, en_ref, emax_ref, gacc_ref, *, ng):
    i = pl.program_id(0)
    emax = emax_ref[...]
    emax = jnp.where(emax == F32(-jnp.inf), F32(0.0), emax)
    idrow = ids_ref[0]
    oht = (idrow == lax.broadcasted_iota(jnp.int32, (ng, B), 0)).astype(F32)
    esel = jnp.sum(oht * emax, axis=0, keepdims=True)
    ex = jnp.exp(en_ref[0] - esel)
    w = oht * ex
    s = _dot(w, d_ref[...])
    denom = jnp.sum(w, axis=1, keepdims=True)
    upd = jnp.concatenate([denom, s], axis=1)

    @pl.when(i == 0)
    def _():
        gacc_ref[...] = jnp.zeros_like(gacc_ref)

    gacc_ref[...] += upd


def _s2sb(data, ids3, en3, emax, rows, ng):
    return pl.pallas_call(
        functools.partial(_s2sb_body, ng=ng),
        grid=(rows // B,),
        in_specs=[_row(32), _sca(), _sca(), _full(ng, 1)],
        out_specs=pl.BlockSpec((ng, 33), lambda i: (0, 0)),
        out_shape=jax.ShapeDtypeStruct((ng, 33), F32),
    )(data, ids3, en3, emax)


def _set2set(data, ids3, p, rows, ng):
    h = jnp.zeros((ng, 32), F32)
    c = jnp.zeros((ng, 32), F32)
    q = jnp.zeros((ng, 32), F32)
    gacc = jnp.zeros((ng, 33), F32)
    for _ in range(3):
        h, c, q = _s2sl(h, c, q, gacc, p, ng)
        en3, emax = _s2sa(data, ids3, q, rows, ng)
        gacc = _s2sb(data, ids3, en3, emax, rows, ng)
    return q, gacc


def _k12_body(qn_ref, gn_ref, qe_ref, ge_ref, u3_ref, w1_ref, b1_ref, g1_ref,
              be1_ref, w2_ref, b2_ref, g2_ref, be2_ref, w3_ref, b3_ref,
              out_ref):
    rn = gn_ref[:, 1:33] / (gn_ref[:, 0:1] + F32(1e-16))
    re = ge_ref[:, 1:33] / (ge_ref[:, 0:1] + F32(1e-16))
    z = jnp.concatenate([qn_ref[...], rn, qe_ref[...], re, u3_ref[...]],
                        axis=1)

    def bn(t, gamma, beta):
        m = jnp.mean(t, axis=0, keepdims=True)
        v = jnp.mean((t - m) ** 2, axis=0, keepdims=True)
        return (t - m) / jnp.sqrt(v + F32(1e-5)) * gamma + beta

    h1 = jax.nn.relu(bn(_dot(z, w1_ref[...]) + b1_ref[...], g1_ref[...],
                        be1_ref[...]))
    h2 = jax.nn.relu(bn(_dot(h1, w2_ref[...]) + b2_ref[...], g2_ref[...],
                        be2_ref[...]))
    out_ref[...] = _dot(h2, w3_ref[...]) + b3_ref[...]


def _k12(qn, gn, qe, ge, u3, hd, ng):
    return pl.pallas_call(
        _k12_body,
        out_shape=jax.ShapeDtypeStruct((ng, 1), F32),
    )(qn, gn, qe, ge, u3, hd['W1'], hd['b1'].reshape(1, -1),
      hd['g1'].reshape(1, -1), hd['be1'].reshape(1, -1), hd['W2'],
      hd['b2'].reshape(1, -1), hd['g2'].reshape(1, -1),
      hd['be2'].reshape(1, -1), hd['W3'], hd['b3'].reshape(1, -1))


def _gather_eb(batch, src):
    return batch[src]


def _gather_gsum(a_s, a_d, c, src, dst, eb):
    return a_s[src] + a_d[dst] + c[eb]


def _scatter_edges(edata, dst, eb, n, ng):
    ones = jnp.ones((edata.shape[0], 1), F32)
    cat = jnp.concatenate([ones, edata], axis=1)
    nodeacc = jax.ops.segment_sum(cat, dst, num_segments=n)
    gacc = jax.ops.segment_sum(cat, eb, num_segments=ng)
    return nodeacc, gacc


def kernel(x, edge_index, edge_attr, state, batch, params):
    n = x.shape[0]
    e = edge_attr.shape[0]
    ng = state.shape[0]
    src = edge_index[0]
    dst = edge_index[1]
    x3 = x.reshape(n // B, 1, B)
    batch3 = batch.reshape(n // B, 1, B)
    attr3 = edge_attr.reshape(e // B, 1, B)

    b1 = params['block1']
    b2 = params['block2']

    w1e_1 = b1['phi_e'][0]['W']
    ws1, wd1, we1, wu1 = (w1e_1[0:32], w1e_1[32:64], w1e_1[64:96],
                          w1e_1[96:128])
    w1e_2 = b2['phi_e'][0]['W']
    ws2, wd2, we2, wu2 = (w1e_2[0:32], w1e_2[32:64], w1e_2[64:96],
                          w1e_2[96:128])

    eb = _gather_eb(batch, src)
    eb3 = eb.reshape(e // B, 1, B)

    v1, a1s, a1d = _k1(x3, params['emb'], b1['node_pre'], ws1, wd1, n)
    u1p, c1 = _kstate(state, b1['state_pre'], wu1, b1['phi_e'][0]['b'])
    gsum1 = _gather_gsum(a1s, a1d, c1, src, dst, eb)
    e1, t2 = _k34(attr3, gsum1, b1['edge_pre'], we1, b1['phi_e'][1]['W'],
                  b1['phi_e'][1]['b'], b2['edge_pre'], we2, e)
    nodeacc1, gacc1 = _scatter_edges(e1, dst, eb, n, ng)
    v2p, a2s, a2d, uvacc1 = _k5(v1, nodeacc1, u1p, batch3, b1['phi_v'],
                                b2['node_pre'], ws2, wd2, n, ng)
    u2 = _kphiu(gacc1, uvacc1, u1p, b1['phi_u'], ng)

    u2p, c2 = _kstate(u2, b2['state_pre'], wu2, b2['phi_e'][0]['b'])
    gsum2 = _gather_gsum(a2s, a2d, c2, src, dst, eb)
    e2 = _k7(t2, gsum2, b2['phi_e'][1]['W'], b2['phi_e'][1]['b'], e)
    nodeacc2, gacc2 = _scatter_edges(e2, dst, eb, n, ng)
    v2, uvacc2 = _k8(v2p, nodeacc2, u2p, batch3, b2['phi_v'], n, ng)
    u3 = _kphiu(gacc2, uvacc2, u2p, b2['phi_u'], ng)

    qn, gn = _set2set(v2, batch3, params['s2s_nodes'], n, ng)
    qe, ge = _set2set(e2, eb3, params['s2s_edges'], e, ng)

    return _k12(qn, gn, qe, ge, u3, params['head'], ng)
